# baseline (speedup 1.0000x reference)
import functools

import jax
import jax.numpy as jnp
from jax import lax
from jax.experimental import pallas as pl
from jax.experimental.pallas import tpu as pltpu
from jax.experimental.pallas import tpu_sc as plsc

_IN_DIM = 1000000
_OUT_DIM = 64
_BATCH = 16384
_TROW = 8

_NC = 2
_NS = 16
_NW = _NC * _NS
_BPW = _BATCH // _NW
_CHUNK = 16
_NCHUNK = _BPW // _CHUNK
_NBUF = 2
_L = 16


def _extract(vec, lane):
    onehot = lax.iota(jnp.int32, _L) == lane
    return jnp.sum(jnp.where(onehot, vec, 0))


@functools.partial(
    pl.kernel,
    mesh=plsc.VectorSubcoreMesh(core_axis_name="c", subcore_axis_name="s"),
    out_type=jax.ShapeDtypeStruct((_NW, _BPW, _OUT_DIM), jnp.float32),
    scratch_types=[
        pltpu.VMEM((_NCHUNK, _CHUNK), jnp.int32),
        pltpu.VMEM((_NBUF, _CHUNK, _TROW, _OUT_DIM), jnp.float32),
        pltpu.VMEM((_BPW, _OUT_DIM), jnp.float32),
        pltpu.SemaphoreType.DMA,
        pltpu.SemaphoreType.DMA,
    ],
    compiler_params=pltpu.CompilerParams(use_tc_tiling_on_sc=True,
                                         needs_layout_passes=False),
)
def _sc_gather(idx_hbm, table_hbm, out_hbm, idx_v, slab_v, rows_v, sem0,
               sem1):
    sems = [sem0, sem1]
    wid = lax.axis_index("s") * _NC + lax.axis_index("c")
    pltpu.sync_copy(idx_hbm.at[wid], idx_v)

    def fire(c):
        b = c % _NBUF

        def body(i, _):
            vec = idx_v[c, pl.ds(lax.shift_left(lax.shift_right_logical(i, 4), 4), _L)]
            v = _extract(vec, lax.bitwise_and(i, _L - 1))
            t = lax.shift_right_logical(v, 3)
            pltpu.async_copy(table_hbm.at[pl.ds(t * _TROW, _TROW)],
                             slab_v.at[b, i], sems[b])
            return 0

        lax.fori_loop(0, _CHUNK, body, 0)

    def wait(c):
        b = c % _NBUF

        def body(i, _):
            pltpu.make_async_copy(table_hbm.at[pl.ds(0, _TROW)],
                                  slab_v.at[b, 0], sems[b]).wait()
            return 0

        lax.fori_loop(0, _CHUNK, body, 0)

    def extract(c):
        b = c % _NBUF

        def body(i, _):
            vec = idx_v[c, pl.ds(lax.shift_left(lax.shift_right_logical(i, 4), 4), _L)]
            v = _extract(vec, lax.bitwise_and(i, _L - 1))
            r = lax.bitwise_and(v, _TROW - 1)
            row = c * _CHUNK + i
            for q in range(_OUT_DIM // _L):
                rows_v[row, pl.ds(q * _L, _L)] = slab_v[b, i, r,
                                                        pl.ds(q * _L, _L)]
            return 0

        lax.fori_loop(0, _CHUNK, body, 0)

    fire(0)
    for c in range(_NCHUNK):
        wait(c)
        if c + 1 < _NCHUNK:
            fire(c + 1)
        extract(c)
    pltpu.sync_copy(rows_v, out_hbm.at[wid])


def kernel(x, ker):
    idx = jnp.reshape(x, (_NW, _NCHUNK, _CHUNK)).astype(jnp.int32)
    out = _sc_gather(idx, ker)
    return jnp.reshape(out, (_BATCH, _OUT_DIM))


# speedup vs baseline: 1.4007x; 1.4007x over previous
import functools

import jax
import jax.numpy as jnp
from jax import lax
from jax.experimental import pallas as pl
from jax.experimental.pallas import tpu as pltpu
from jax.experimental.pallas import tpu_sc as plsc

_IN_DIM = 1000000
_OUT_DIM = 64
_BATCH = 16384
_TROW = 8
_S = 16

_NC = 2
_NS = 16
_NW = _NC * _NS
_BPW = _BATCH // _NW
_CHUNK = 16
_NCHUNK = _BPW // _CHUNK
_NBUF = 2
_L = 16


def _extract(vec, lane):
    onehot = lax.iota(jnp.int32, _L) == lane
    return jnp.sum(jnp.where(onehot, vec, 0))


@functools.partial(
    pl.kernel,
    mesh=plsc.VectorSubcoreMesh(core_axis_name="c", subcore_axis_name="s"),
    out_type=jax.ShapeDtypeStruct((_NW, _BPW, _OUT_DIM), jnp.float32),
    scratch_types=[
        pltpu.VMEM((_NCHUNK, _CHUNK), jnp.int32),
        pltpu.VMEM((_NBUF, _CHUNK, _TROW, _OUT_DIM), jnp.float32),
        pltpu.VMEM((_BPW, _OUT_DIM), jnp.float32),
        pltpu.SemaphoreType.DMA,
        pltpu.SemaphoreType.DMA,
    ],
    compiler_params=pltpu.CompilerParams(use_tc_tiling_on_sc=True,
                                         needs_layout_passes=False),
)
def _sc_gather(idx_hbm, table_hbm, out_hbm, idx_v, slab_v, rows_v, sem0,
               sem1):
    sems = [sem0, sem1]
    wid = lax.axis_index("s") * _NC + lax.axis_index("c")
    pltpu.sync_copy(idx_hbm.at[wid], idx_v)

    def fire(c):
        b = c % _NBUF

        def body(i, _):
            vec = idx_v[c, pl.ds(lax.shift_left(lax.shift_right_logical(i, 4), 4), _L)]
            v = _extract(vec, lax.bitwise_and(i, _L - 1))
            t = lax.shift_right_logical(v, 4)
            h = lax.bitwise_and(lax.shift_right_logical(v, 3), _S // _TROW - 1)
            pltpu.async_copy(table_hbm.at[t, pl.ds(h * _TROW, _TROW)],
                             slab_v.at[b, i], sems[b])
            return 0

        lax.fori_loop(0, _CHUNK, body, 0)

    def wait(c):
        b = c % _NBUF

        def body(i, _):
            pltpu.make_async_copy(table_hbm.at[0, pl.ds(0, _TROW)],
                                  slab_v.at[b, 0], sems[b]).wait()
            return 0

        lax.fori_loop(0, _CHUNK, body, 0)

    def extract(c):
        b = c % _NBUF

        def body(i, _):
            vec = idx_v[c, pl.ds(lax.shift_left(lax.shift_right_logical(i, 4), 4), _L)]
            v = _extract(vec, lax.bitwise_and(i, _L - 1))
            r = lax.bitwise_and(v, _TROW - 1)
            row = c * _CHUNK + i
            for q in range(_OUT_DIM // _L):
                rows_v[row, pl.ds(q * _L, _L)] = slab_v[b, i, r,
                                                        pl.ds(q * _L, _L)]
            return 0

        lax.fori_loop(0, _CHUNK, body, 0)

    fire(0)
    for c in range(_NCHUNK):
        wait(c)
        if c + 1 < _NCHUNK:
            fire(c + 1)
        extract(c)
    pltpu.sync_copy(rows_v, out_hbm.at[wid])


def kernel(x, ker):
    idx = jnp.reshape(x, (_NW, _NCHUNK, _CHUNK)).astype(jnp.int32)
    table = jnp.reshape(ker, (_IN_DIM // _S, _S, _OUT_DIM))
    out = _sc_gather(idx, table)
    return jnp.reshape(out, (_BATCH, _OUT_DIM))


# speedup vs baseline: 2.4146x; 1.7238x over previous
import functools

import jax
import jax.numpy as jnp
from jax import lax
from jax.experimental import pallas as pl
from jax.experimental.pallas import tpu as pltpu
from jax.experimental.pallas import tpu_sc as plsc

_IN_DIM = 1000000
_OUT_DIM = 64
_BATCH = 16384

_NC = 2
_NS = 16
_NW = _NC * _NS
_L = 16

_LANE = 128
_NTILE = -(-_IN_DIM // _LANE)
_BASE_T = _NTILE // _NW
_EXTRA = _NTILE - _BASE_T * _NW
_PIECE_T = 5
_PIECE_C = _PIECE_T * _LANE
_NPIECE = -(-(_BASE_T + 1) // _PIECE_T)
_MAX_START = (_IN_DIM - _PIECE_C) // _LANE
_TAIL_C = _NTILE * _LANE - _PIECE_C - _MAX_START * _LANE
_TAIL0 = _IN_DIM - (_IN_DIM % _LANE)
_RING = 32
_NIDX_V = _BATCH // _L


def _extract(vec, lane):
    onehot = lax.iota(jnp.int32, _L) == lane
    return jnp.sum(jnp.where(onehot, vec, 0))


@functools.partial(
    pl.kernel,
    mesh=plsc.VectorSubcoreMesh(core_axis_name="c", subcore_axis_name="s"),
    out_type=jax.ShapeDtypeStruct((_BATCH * _OUT_DIM,), jnp.float32),
    scratch_types=[
        pltpu.VMEM((128, 128), jnp.int32),
        pltpu.VMEM((_BATCH + _L,), jnp.int32),
        pltpu.VMEM((2, _OUT_DIM, _PIECE_C), jnp.float32),
        pltpu.VMEM((_RING, _OUT_DIM), jnp.float32),
        pltpu.VMEM((_L,), jnp.int32),
        pltpu.VMEM((_OUT_DIM, _IN_DIM - _TAIL0), jnp.float32),
        pltpu.SemaphoreType.DMA,
        pltpu.SemaphoreType.DMA,
        pltpu.SemaphoreType.DMA,
    ],
    compiler_params=pltpu.CompilerParams(use_tc_tiling_on_sc=True,
                                         needs_layout_passes=False),
)
def _sc_gather(idx_hbm, tablet_hbm, out_hbm, idx_v, hit_v, slab_v, stage_v,
               tmp_v, tail_v, psem0, psem1, osem):
    psems = [psem0, psem1]
    wid = lax.axis_index("s") * _NC + lax.axis_index("c")
    b0 = _BASE_T * wid + jnp.minimum(wid, _EXTRA)
    bt = _BASE_T + jnp.where(wid < _EXTRA, 1, 0)
    c_lo = b0 * _LANE
    c_hi = (b0 + bt) * _LANE
    iota = lax.iota(jnp.int32, _L)

    pltpu.sync_copy(idx_hbm, idx_v)

    def pre(k, m):
        vec = idx_v[lax.shift_right_logical(k, 3),
                    pl.ds(lax.bitwise_and(k, 7) * _L, _L)]
        inband = jnp.logical_and(vec >= c_lo, vec < c_hi)
        packed = lax.bitwise_or(lax.shift_left(vec - c_lo, 14), k * _L + iota)
        plsc.store_compressed(hit_v.at[pl.ds(m, _L)], packed, mask=inband)
        return m + jnp.sum(jnp.where(inband, 1, 0))

    m = lax.fori_loop(0, _NIDX_V, pre, 0)
    nvec = lax.shift_right_logical(m + _L - 1, 4)

    def fire(p):
        t0 = b0 + jnp.minimum(_PIECE_T * p, bt - _PIECE_T)
        t0 = jnp.minimum(t0, _MAX_START)
        pltpu.async_copy(tablet_hbm.at[:, pl.ds(t0 * _LANE, _PIECE_C)],
                         slab_v.at[p % 2], psems[p % 2])
        return t0 * _LANE

    def drain_out(n):
        def w(_, c):
            pltpu.make_async_copy(stage_v.at[0],
                                  out_hbm.at[pl.ds(0, _OUT_DIM)], osem).wait()
            return c

        lax.fori_loop(0, n, w, 0)

    def scan_piece(buf, p0, p1, state):
        def scan(j, st):
            hvec = hit_v[pl.ds(j * _L, _L)]
            cols = lax.shift_right_logical(hvec, 14) + c_lo
            valid = (j * _L + iota) < m
            inp = jnp.logical_and(jnp.logical_and(cols >= p0, cols < p1),
                                  valid)
            packed2 = lax.bitwise_or(lax.shift_left(cols - p0, 14),
                                     lax.bitwise_and(hvec, 16383))
            plsc.store_compressed(tmp_v.at[pl.ds(0, _L)], packed2, mask=inp)
            cnt = jnp.sum(jnp.where(inp, 1, 0))

            def hit(h, st2):
                outst, ring = st2
                hv = _extract(tmp_v[pl.ds(0, _L)], h)
                c = lax.shift_right_logical(hv, 14)
                pos = lax.bitwise_and(hv, 16383)
                csplat = jnp.full((_L,), c, jnp.int32)
                for q in range(_OUT_DIM // _L):
                    stage_v[ring, pl.ds(q * _L, _L)] = plsc.load_gather(
                        buf, [iota + q * _L, csplat])
                pltpu.async_copy(stage_v.at[ring],
                                 out_hbm.at[pl.ds(pos * _OUT_DIM, _OUT_DIM)],
                                 osem)
                outst = outst + 1
                wrap = ring + 1 == _RING

                @pl.when(wrap)
                def _():
                    drain_out(outst)

                return (jnp.where(wrap, 0, outst),
                        jnp.where(wrap, 0, ring + 1))

            return lax.fori_loop(0, cnt, hit, st)

        return lax.fori_loop(0, nvec, scan, state)

    state = (jnp.int32(0), jnp.int32(0))
    p0 = fire(0)
    starts = [p0]
    for p in range(_NPIECE):
        pltpu.make_async_copy(tablet_hbm.at[:, pl.ds(0, _PIECE_C)],
                              slab_v.at[p % 2], psems[p % 2]).wait()
        if p + 1 < _NPIECE:
            starts.append(fire(p + 1))
        state = scan_piece(slab_v.at[p % 2], starts[p], starts[p] + _PIECE_C,
                           state)

    @pl.when(c_hi > _TAIL0)
    def _():
        pltpu.sync_copy(tablet_hbm.at[:, pl.ds(_TAIL0, _IN_DIM - _TAIL0)],
                        tail_v)

    tail_state = scan_piece(tail_v, jnp.int32(_TAIL0), jnp.int32(_IN_DIM),
                            state)
    drain_out(tail_state[0])


def kernel(x, ker):
    idx = jnp.reshape(x, (128, 128)).astype(jnp.int32)
    out = _sc_gather(idx, ker.T)
    return jnp.reshape(out, (_BATCH, _OUT_DIM))
